# baseline (speedup 1.0000x reference)
import functools

import jax
import jax.numpy as jnp
from jax import lax
from jax.experimental import pallas as pl
from jax.experimental.pallas import tpu as pltpu
from jax.experimental.pallas import tpu_sc as plsc

_NPOINT = 512
_RADIUS = 0.2
_K = 32
_DOUT = 256
_BIGF = 3.0e38


def _fps_body(x_ref, y_ref, z_ref, nx_ref, ny_ref, nz_ref, dm_ref):
    B, N = x_ref.shape
    x = x_ref[...]
    y = y_ref[...]
    z = z_ref[...]
    lanes = lax.broadcasted_iota(jnp.int32, (B, N), 1)

    fx0 = x[:, 0:1]
    fy0 = y[:, 0:1]
    fz0 = z[:, 0:1]
    dx = x - fx0
    dy = y - fy0
    dz = z - fz0
    dm_ref[...] = dx * dx + dy * dy + dz * dz

    tcols = lax.broadcasted_iota(jnp.int32, (B, 128), 1)

    def body(i, carry):
        tx, ty, tz = carry
        d = dm_ref[...]
        m = jnp.max(d, axis=1, keepdims=True)
        far = jnp.min(jnp.where(d == m, lanes, N), axis=1, keepdims=True)
        sel = lanes == far
        fx = jnp.sum(jnp.where(sel, x, 0.0), axis=1, keepdims=True)
        fy = jnp.sum(jnp.where(sel, y, 0.0), axis=1, keepdims=True)
        fz = jnp.sum(jnp.where(sel, z, 0.0), axis=1, keepdims=True)
        ddx = x - fx
        ddy = y - fy
        ddz = z - fz
        nd = ddx * ddx + ddy * ddy + ddz * ddz
        dm_ref[...] = jnp.minimum(d, nd)
        here = tcols == (i % 128)
        return (jnp.where(here, fx, tx), jnp.where(here, fy, ty),
                jnp.where(here, fz, tz))

    zt = jnp.zeros((B, 128), jnp.float32)
    for p in range(_NPOINT // 128):
        lo = max(p * 128, 1)
        tx, ty, tz = lax.fori_loop(lo, (p + 1) * 128, body, (zt, zt, zt))
        if p == 0:
            first = tcols == 0
            tx = jnp.where(first, fx0, tx)
            ty = jnp.where(first, fy0, ty)
            tz = jnp.where(first, fz0, tz)
        nx_ref[:, p * 128:(p + 1) * 128] = tx
        ny_ref[:, p * 128:(p + 1) * 128] = ty
        nz_ref[:, p * 128:(p + 1) * 128] = tz


def _run_fps(x, y, z):
    B, N = x.shape
    out = jax.ShapeDtypeStruct((B, _NPOINT), jnp.float32)
    return pl.pallas_call(
        _fps_body,
        out_shape=[out, out, out],
        scratch_shapes=[pltpu.VMEM((B, N), jnp.float32)],
    )(x, y, z)


_SBQ = 128


_L = 16
_BIGI = 1 << 30


def _blane(v, lane):
    idx = jnp.full((_L,), lane, jnp.int32)
    return lax.gather(
        v, idx[:, None],
        lax.GatherDimensionNumbers(offset_dims=(), collapsed_slice_dims=(0,),
                                   start_index_map=(0,)),
        (1,), mode=lax.GatherScatterMode.PROMISE_IN_BOUNDS)


_CAP = 1024
_PCAP = 96


def _run_ball_query_sc(xp, yp, zp, nxp, nyp, nzp):
    B, N = xp.shape
    rows_per_w = (B * _NPOINT) // _NW
    per_batch = _NPOINT // rows_per_w
    ngroup = rows_per_w // _L
    nchunk = N // _L
    mesh = plsc.VectorSubcoreMesh(core_axis_name="c", subcore_axis_name="s")
    r2 = _RADIUS ** 2

    @functools.partial(
        pl.kernel,
        out_type=jax.ShapeDtypeStruct((B * _NPOINT * _K,), jnp.int32),
        mesh=mesh,
        scratch_types=[
            pltpu.VMEM((N,), jnp.float32),
            pltpu.VMEM((N,), jnp.float32),
            pltpu.VMEM((N,), jnp.float32),
            pltpu.VMEM((rows_per_w,), jnp.float32),
            pltpu.VMEM((rows_per_w,), jnp.float32),
            pltpu.VMEM((rows_per_w,), jnp.float32),
            pltpu.VMEM((_L * _CAP + _L,), jnp.float32),
            pltpu.VMEM((_L * _CAP + _L,), jnp.int32),
            pltpu.VMEM((_L * _PCAP,), jnp.int32),
            pltpu.VMEM((rows_per_w * _K,), jnp.int32),
            pltpu.SemaphoreType.DMA,
        ],
        compiler_params=pltpu.CompilerParams(needs_layout_passes=False),
    )
    def k(x_hbm, y_hbm, z_hbm, cx_hbm, cy_hbm, cz_hbm, out_hbm,
          xv, yv, zv, cxv, cyv, czv, dbuf, ibuf, pbuf, obuf, sem):
        iota = lax.iota(jnp.int32, _L)
        lane0 = iota == 0
        c = lax.axis_index("c")
        s = lax.axis_index("s")
        wid = s * 2 + c
        b = wid // per_batch
        soff = (wid % per_batch) * rows_per_w
        pltpu.sync_copy(x_hbm.at[b], xv)
        pltpu.sync_copy(y_hbm.at[b], yv)
        pltpu.sync_copy(z_hbm.at[b], zv)
        pltpu.sync_copy(cx_hbm.at[b, pl.ds(soff, rows_per_w)], cxv)
        pltpu.sync_copy(cy_hbm.at[b, pl.ds(soff, rows_per_w)], cyv)
        pltpu.sync_copy(cz_hbm.at[b, pl.ds(soff, rows_per_w)], czv)
        gbase = b * N
        lane_base = iota * _CAP
        plane_base = iota * _PCAP

        def group_body(g, _):
            cx16 = cxv[pl.ds(g * _L, _L)]
            cy16 = cyv[pl.ds(g * _L, _L)]
            cz16 = czv[pl.ds(g * _L, _L)]

            def cand_chunk(t, offv):
                xa = xv[pl.ds(t * _L, _L)]
                ya = yv[pl.ds(t * _L, _L)]
                za = zv[pl.ds(t * _L, _L)]
                for l2 in range(_L):
                    xs = _blane(xa, l2)
                    ys = _blane(ya, l2)
                    zs = _blane(za, l2)
                    dx = cx16 - xs
                    dy = cy16 - ys
                    dz = cz16 - zs
                    d = dx * dx + dy * dy + dz * dz
                    m = d <= r2
                    pos = lane_base + offv
                    jsplat = jnp.full((_L,), t * _L + l2, jnp.int32)
                    plsc.store_scatter(dbuf, [pos], d, mask=m)
                    plsc.store_scatter(ibuf, [pos], jsplat, mask=m)
                    offv = offv + m.astype(jnp.int32)
                return offv

            offv = lax.fori_loop(0, nchunk, cand_chunk,
                                 jnp.zeros((_L,), jnp.int32))

            def pad_chunk(t, offp):
                xa = xv[pl.ds(t * _L, _L)]
                ya = yv[pl.ds(t * _L, _L)]
                za = zv[pl.ds(t * _L, _L)]
                for l2 in range(_L):
                    xs = _blane(xa, l2)
                    ys = _blane(ya, l2)
                    zs = _blane(za, l2)
                    dx = cx16 - xs
                    dy = cy16 - ys
                    dz = cz16 - zs
                    d = dx * dx + dy * dy + dz * dz
                    m = d > r2
                    pos = plane_base + offp
                    jsplat = jnp.full((_L,), t * _L + l2, jnp.int32)
                    plsc.store_scatter(pbuf, [pos], jsplat, mask=m)
                    offp = offp + m.astype(jnp.int32)
                return offp

            lax.fori_loop(0, 4, pad_chunk, jnp.zeros((_L,), jnp.int32))

            def row_fin(r2i, _):
                r = g * _L + r2i
                cnt = jnp.max(_blane(offv, r2i))
                rbase = r2i * _CAP
                pbase = r2i * _PCAP
                obase = r * _K
                plsc.store_scatter(dbuf, [rbase + cnt + iota],
                                   jnp.full((_L,), _BIGF, jnp.float32))

                def few():
                    iv0 = ibuf[pl.ds(rbase, _L)] + gbase
                    iv1 = ibuf[pl.ds(rbase + _L, _L)] + gbase
                    obuf[pl.ds(obase, _L)] = iv0
                    obuf[pl.ds(obase + _L, _L)] = iv1
                    for t in range(2):
                        pv = pbuf[pl.ds(pbase + t * _L, _L)] + gbase
                        ppos = cnt + iota + t * _L
                        plsc.store_scatter(obuf, [ppos + obase], pv,
                                           mask=ppos < _K)

                def many():
                    nch = (cnt + _L) // _L

                    def extract(kk, _):
                        def scan(j, carry):
                            vbest, vpos = carry
                            v = dbuf[pl.ds(rbase + j * _L, _L)]
                            better = v < vbest
                            return (jnp.where(better, v, vbest),
                                    jnp.where(better, j * _L + iota, vpos))

                        vbest, vpos = lax.fori_loop(
                            0, nch, scan,
                            (jnp.full((_L,), _BIGF, jnp.float32),
                             jnp.full((_L,), _BIGI, jnp.int32)))
                        m = jnp.min(vbest)
                        p = jnp.min(jnp.where(vbest == m, vpos, _BIGI))
                        psplat = jnp.full((_L,), p, jnp.int32) + rbase
                        gv = plsc.load_gather(ibuf, [psplat]) + gbase
                        plsc.store_scatter(
                            obuf, [jnp.full((_L,), obase + kk, jnp.int32)],
                            gv, mask=lane0)
                        plsc.store_scatter(
                            dbuf, [psplat],
                            jnp.full((_L,), _BIGF, jnp.float32), mask=lane0)
                        return 0

                    lax.fori_loop(0, _K, extract, 0)

                lax.cond(cnt > _K, many, few)
                return 0

            lax.fori_loop(0, _L, row_fin, 0)
            return 0

        lax.fori_loop(0, ngroup, group_body, 0)
        pltpu.sync_copy(obuf, out_hbm.at[pl.ds(wid * rows_per_w * _K,
                                               rows_per_w * _K)])

    return k(xp, yp, zp, nxp, nyp, nzp)


_NW = 32
_GROWS = 128


def _run_gather(table, idx2d):
    n_idx_rows = idx2d.shape[0]
    rows_per_w = n_idx_rows // _NW
    total = n_idx_rows * _GROWS
    width = table.shape[1]
    mesh = plsc.VectorSubcoreMesh(core_axis_name="c", subcore_axis_name="s")

    @functools.partial(
        pl.kernel,
        out_type=jax.ShapeDtypeStruct((total, width), jnp.float32),
        mesh=mesh,
        scratch_types=[
            pltpu.VMEM((rows_per_w, _GROWS), jnp.int32),
            pltpu.VMEM((_GROWS, width), jnp.float32),
            pltpu.SemaphoreType.DMA,
        ],
    )
    def k(table_hbm, idx_hbm, out_hbm, idx_v, rows_v, sem):
        c = lax.axis_index("c")
        s = lax.axis_index("s")
        wid = s * 2 + c
        base = wid * rows_per_w
        pltpu.sync_copy(idx_hbm.at[pl.ds(base, rows_per_w)], idx_v)

        def body(j, _):
            pltpu.async_copy(table_hbm.at[idx_v.at[j]], rows_v, sem).wait()
            pltpu.sync_copy(rows_v, out_hbm.at[pl.ds((base + j) * _GROWS, _GROWS)])
            return 0

        lax.fori_loop(0, rows_per_w, body, 0, unroll=False)

    return k(table, idx2d)


_SMLP = 32
_RML = _SMLP * _K


def _mlp_body(g_ref, cxyz_ref, w1a_ref, w1a3_ref, b1a_ref, w1b_ref, b1b_ref,
              wwaf_ref, wwa3_ref, bwa_ref, wwb_ref, bwb_ref, out_ref):
    g = g_ref[...]
    cxyz = cxyz_ref[...]
    ctr1 = jnp.dot(cxyz, w1a3_ref[...], preferred_element_type=jnp.float32)
    ctr2 = jnp.dot(cxyz, wwa3_ref[...], preferred_element_type=jnp.float32)

    def rep(a):
        return jnp.reshape(
            jnp.broadcast_to(a[:, None, :], (_SMLP, _K, a.shape[-1])),
            (_RML, a.shape[-1]))

    bf = jnp.bfloat16
    t1 = jnp.dot(g.astype(bf), w1a_ref[...].astype(bf),
                 preferred_element_type=jnp.float32)
    h = jnp.maximum(t1 + b1a_ref[...] - rep(ctr1), 0.0)
    fp = jnp.maximum(
        jnp.dot(h.astype(bf), w1b_ref[...].astype(bf),
                preferred_element_type=jnp.float32) + b1b_ref[...], 0.0)
    fm = jnp.sum(jnp.reshape(fp, (_SMLP, _K, _DOUT)), axis=1) * (1.0 / _K)
    xyzp = g[:, 0:8]
    t2 = (jnp.dot((fp - rep(fm)).astype(bf), wwaf_ref[...].astype(bf),
                  preferred_element_type=jnp.float32)
          + jnp.dot(xyzp, wwa3_ref[...], preferred_element_type=jnp.float32)
          - rep(ctr2))
    hw = jnp.maximum(t2 + bwa_ref[...], 0.0)
    logit = jnp.dot(hw.astype(bf), wwb_ref[...].astype(bf),
                    preferred_element_type=jnp.float32) + bwb_ref[...]
    alpha = jax.nn.sigmoid(logit)
    out_ref[...] = jnp.sum(jnp.reshape(alpha * fp, (_SMLP, _K, _DOUT)), axis=1)


def _run_mlp(g, cxyz, w1a, w1a3, b1a, w1b, b1b, wwaf, wwa3, bwa, wwb, bwb):
    n_rows = g.shape[0]
    grid = (n_rows // _RML,)

    def const(shape):
        return pl.BlockSpec(shape, lambda i: tuple(0 for _ in shape))

    return pl.pallas_call(
        _mlp_body,
        grid=grid,
        in_specs=[
            pl.BlockSpec((_RML, 128), lambda i: (i, 0)),
            pl.BlockSpec((_SMLP, 8), lambda i: (i, 0)),
            const(w1a.shape), const(w1a3.shape), const(b1a.shape),
            const(w1b.shape), const(b1b.shape), const(wwaf.shape),
            const(wwa3.shape), const(bwa.shape), const(wwb.shape),
            const(bwb.shape),
        ],
        out_specs=pl.BlockSpec((_SMLP, _DOUT), lambda i: (i, 0)),
        out_shape=jax.ShapeDtypeStruct((n_rows // _K, _DOUT), jnp.float32),
    )(g, cxyz, w1a, w1a3, b1a, w1b, b1b, wwaf, wwa3, bwa, wwb, bwb)


def kernel(xyz, features, W1a, b1a, W1b, b1b, Wwa, bwa, Wwb, bwb):
    B, N, _ = xyz.shape
    C = features.shape[-1]
    x = xyz[:, :, 0]
    y = xyz[:, :, 1]
    z = xyz[:, :, 2]

    nx, ny, nz = _run_fps(x, y, z)
    new_xyz = jnp.stack([nx, ny, nz], axis=-1)

    idx_flat = _run_ball_query_sc(x, y, z, nx, ny, nz)
    idx2d = idx_flat.reshape(-1, _GROWS)

    table = jnp.concatenate(
        [xyz, features, jnp.zeros((B, N, 128 - 3 - C), jnp.float32)],
        axis=-1).reshape(B * N, 128)
    g = _run_gather(table, idx2d)

    cxyz = jnp.concatenate(
        [new_xyz, jnp.zeros((B, _NPOINT, 5), jnp.float32)], axis=-1
    ).reshape(B * _NPOINT, 8)
    w1a = jnp.concatenate([W1a, jnp.zeros((128 - W1a.shape[0], _DOUT), jnp.float32)])
    w1a3 = jnp.concatenate([W1a[:3], jnp.zeros((5, _DOUT), jnp.float32)])
    wwa3 = jnp.concatenate([Wwa[:3], jnp.zeros((5, _DOUT), jnp.float32)])
    wwaf = Wwa[3:]
    f_out = _run_mlp(g, cxyz, w1a, w1a3, b1a[None, :], W1b, b1b[None, :],
                     wwaf, wwa3, bwa[None, :], Wwb, bwb[None, :])
    return (new_xyz, f_out.reshape(B, _NPOINT, _DOUT))


# speedup vs baseline: 1.0186x; 1.0186x over previous
import functools

import jax
import jax.numpy as jnp
from jax import lax
from jax.experimental import pallas as pl
from jax.experimental.pallas import tpu as pltpu
from jax.experimental.pallas import tpu_sc as plsc

_NPOINT = 512
_RADIUS = 0.2
_K = 32
_DOUT = 256
_BIGF = 3.0e38


def _fps_body(x_ref, y_ref, z_ref, nx_ref, ny_ref, nz_ref, dm_ref):
    B, N = x_ref.shape
    x = x_ref[...]
    y = y_ref[...]
    z = z_ref[...]
    lanes = lax.broadcasted_iota(jnp.int32, (B, N), 1)

    fx0 = x[:, 0:1]
    fy0 = y[:, 0:1]
    fz0 = z[:, 0:1]
    dx = x - fx0
    dy = y - fy0
    dz = z - fz0
    dm_ref[...] = dx * dx + dy * dy + dz * dz

    tcols = lax.broadcasted_iota(jnp.int32, (B, 128), 1)

    def body(i, carry):
        tx, ty, tz = carry
        d = dm_ref[...]
        m = jnp.max(d, axis=1, keepdims=True)
        far = jnp.min(jnp.where(d == m, lanes, N), axis=1, keepdims=True)
        sel = lanes == far
        fx = jnp.sum(jnp.where(sel, x, 0.0), axis=1, keepdims=True)
        fy = jnp.sum(jnp.where(sel, y, 0.0), axis=1, keepdims=True)
        fz = jnp.sum(jnp.where(sel, z, 0.0), axis=1, keepdims=True)
        ddx = x - fx
        ddy = y - fy
        ddz = z - fz
        nd = ddx * ddx + ddy * ddy + ddz * ddz
        dm_ref[...] = jnp.minimum(d, nd)
        here = tcols == (i % 128)
        return (jnp.where(here, fx, tx), jnp.where(here, fy, ty),
                jnp.where(here, fz, tz))

    zt = jnp.zeros((B, 128), jnp.float32)
    for p in range(_NPOINT // 128):
        lo = max(p * 128, 1)
        tx, ty, tz = lax.fori_loop(lo, (p + 1) * 128, body, (zt, zt, zt))
        if p == 0:
            first = tcols == 0
            tx = jnp.where(first, fx0, tx)
            ty = jnp.where(first, fy0, ty)
            tz = jnp.where(first, fz0, tz)
        nx_ref[:, p * 128:(p + 1) * 128] = tx
        ny_ref[:, p * 128:(p + 1) * 128] = ty
        nz_ref[:, p * 128:(p + 1) * 128] = tz


def _run_fps(x, y, z):
    B, N = x.shape
    out = jax.ShapeDtypeStruct((B, _NPOINT), jnp.float32)
    return pl.pallas_call(
        _fps_body,
        out_shape=[out, out, out],
        scratch_shapes=[pltpu.VMEM((B, N), jnp.float32)],
    )(x, y, z)


_SBQ = 128


_L = 16
_BIGI = 1 << 30


def _blane(v, lane):
    idx = jnp.full((_L,), lane, jnp.int32)
    return lax.gather(
        v, idx[:, None],
        lax.GatherDimensionNumbers(offset_dims=(), collapsed_slice_dims=(0,),
                                   start_index_map=(0,)),
        (1,), mode=lax.GatherScatterMode.PROMISE_IN_BOUNDS)


_CAP = 1024
_PCAP = 96


def _run_ball_query_sc(xp, yp, zp, nxp, nyp, nzp):
    B, N = xp.shape
    rows_per_w = (B * _NPOINT) // _NW
    per_batch = _NPOINT // rows_per_w
    ngroup = rows_per_w // _L
    nchunk = N // _L
    mesh = plsc.VectorSubcoreMesh(core_axis_name="c", subcore_axis_name="s")
    r2 = _RADIUS ** 2

    @functools.partial(
        pl.kernel,
        out_type=jax.ShapeDtypeStruct((B * _NPOINT * _K,), jnp.int32),
        mesh=mesh,
        scratch_types=[
            pltpu.VMEM((N,), jnp.float32),
            pltpu.VMEM((N,), jnp.float32),
            pltpu.VMEM((N,), jnp.float32),
            pltpu.VMEM((rows_per_w,), jnp.float32),
            pltpu.VMEM((rows_per_w,), jnp.float32),
            pltpu.VMEM((rows_per_w,), jnp.float32),
            pltpu.VMEM((_L * _CAP + _L,), jnp.float32),
            pltpu.VMEM((_L * _CAP + _L,), jnp.int32),
            pltpu.VMEM((_L * _PCAP,), jnp.int32),
            pltpu.VMEM((rows_per_w * _K,), jnp.int32),
            pltpu.SemaphoreType.DMA,
        ],
        compiler_params=pltpu.CompilerParams(needs_layout_passes=False),
    )
    def k(x_hbm, y_hbm, z_hbm, cx_hbm, cy_hbm, cz_hbm, out_hbm,
          xv, yv, zv, cxv, cyv, czv, dbuf, ibuf, pbuf, obuf, sem):
        iota = lax.iota(jnp.int32, _L)
        lane0 = iota == 0
        c = lax.axis_index("c")
        s = lax.axis_index("s")
        wid = s * 2 + c
        b = wid // per_batch
        soff = (wid % per_batch) * rows_per_w
        pltpu.sync_copy(x_hbm.at[b], xv)
        pltpu.sync_copy(y_hbm.at[b], yv)
        pltpu.sync_copy(z_hbm.at[b], zv)
        pltpu.sync_copy(cx_hbm.at[b, pl.ds(soff, rows_per_w)], cxv)
        pltpu.sync_copy(cy_hbm.at[b, pl.ds(soff, rows_per_w)], cyv)
        pltpu.sync_copy(cz_hbm.at[b, pl.ds(soff, rows_per_w)], czv)
        gbase = b * N
        lane_base = iota * _CAP
        plane_base = iota * _PCAP

        def group_body(g, _):
            cx16 = cxv[pl.ds(g * _L, _L)]
            cy16 = cyv[pl.ds(g * _L, _L)]
            cz16 = czv[pl.ds(g * _L, _L)]

            def cand_chunk(t, offv):
                xa = xv[pl.ds(t * _L, _L)]
                ya = yv[pl.ds(t * _L, _L)]
                za = zv[pl.ds(t * _L, _L)]
                for l2 in range(_L):
                    xs = _blane(xa, l2)
                    ys = _blane(ya, l2)
                    zs = _blane(za, l2)
                    dx = cx16 - xs
                    dy = cy16 - ys
                    dz = cz16 - zs
                    d = dx * dx + dy * dy + dz * dz
                    m = d <= r2
                    pos = lane_base + offv
                    jsplat = jnp.full((_L,), t * _L + l2, jnp.int32)
                    plsc.store_scatter(dbuf, [pos], d, mask=m)
                    plsc.store_scatter(ibuf, [pos], jsplat, mask=m)
                    offv = offv + m.astype(jnp.int32)
                return offv

            offv = lax.fori_loop(0, nchunk, cand_chunk,
                                 jnp.zeros((_L,), jnp.int32))

            def pad_chunk(t, offp):
                xa = xv[pl.ds(t * _L, _L)]
                ya = yv[pl.ds(t * _L, _L)]
                za = zv[pl.ds(t * _L, _L)]
                for l2 in range(_L):
                    xs = _blane(xa, l2)
                    ys = _blane(ya, l2)
                    zs = _blane(za, l2)
                    dx = cx16 - xs
                    dy = cy16 - ys
                    dz = cz16 - zs
                    d = dx * dx + dy * dy + dz * dz
                    m = d > r2
                    pos = plane_base + offp
                    jsplat = jnp.full((_L,), t * _L + l2, jnp.int32)
                    plsc.store_scatter(pbuf, [pos], jsplat, mask=m)
                    offp = offp + m.astype(jnp.int32)
                return offp

            lax.fori_loop(0, 4, pad_chunk, jnp.zeros((_L,), jnp.int32))

            def row_fin(r2i, _):
                r = g * _L + r2i
                cnt = jnp.max(_blane(offv, r2i))
                rbase = r2i * _CAP
                pbase = r2i * _PCAP
                obase = r * _K
                plsc.store_scatter(dbuf, [rbase + cnt + iota],
                                   jnp.full((_L,), _BIGF, jnp.float32))

                def few():
                    iv0 = ibuf[pl.ds(rbase, _L)] + gbase
                    iv1 = ibuf[pl.ds(rbase + _L, _L)] + gbase
                    obuf[pl.ds(obase, _L)] = iv0
                    obuf[pl.ds(obase + _L, _L)] = iv1
                    for t in range(2):
                        pv = pbuf[pl.ds(pbase + t * _L, _L)] + gbase
                        ppos = cnt + iota + t * _L
                        plsc.store_scatter(obuf, [ppos + obase], pv,
                                           mask=ppos < _K)

                def many():
                    nch = (cnt + _L) // _L

                    def extract(kk, _):
                        def scan(j, carry):
                            vbest, vpos = carry
                            v = dbuf[pl.ds(rbase + j * _L, _L)]
                            better = v < vbest
                            return (jnp.where(better, v, vbest),
                                    jnp.where(better, j * _L + iota, vpos))

                        vbest, vpos = lax.fori_loop(
                            0, nch, scan,
                            (jnp.full((_L,), _BIGF, jnp.float32),
                             jnp.full((_L,), _BIGI, jnp.int32)))
                        m = jnp.min(vbest)
                        p = jnp.min(jnp.where(vbest == m, vpos, _BIGI))
                        psplat = jnp.full((_L,), p, jnp.int32) + rbase
                        gv = plsc.load_gather(ibuf, [psplat]) + gbase
                        plsc.store_scatter(
                            obuf, [jnp.full((_L,), obase + kk, jnp.int32)],
                            gv, mask=lane0)
                        plsc.store_scatter(
                            dbuf, [psplat],
                            jnp.full((_L,), _BIGF, jnp.float32), mask=lane0)
                        return 0

                    lax.fori_loop(0, _K, extract, 0)

                lax.cond(cnt > _K, many, few)
                return 0

            lax.fori_loop(0, _L, row_fin, 0)
            return 0

        lax.fori_loop(0, ngroup, group_body, 0)
        pltpu.sync_copy(obuf, out_hbm.at[pl.ds(wid * rows_per_w * _K,
                                               rows_per_w * _K)])

    return k(xp, yp, zp, nxp, nyp, nzp)


_NW = 32
_GROWS = 128


def _run_gather(table, idx2d):
    n_idx_rows = idx2d.shape[0]
    rows_per_w = n_idx_rows // _NW
    total = n_idx_rows * _GROWS
    width = table.shape[1]
    mesh = plsc.VectorSubcoreMesh(core_axis_name="c", subcore_axis_name="s")

    @functools.partial(
        pl.kernel,
        out_type=jax.ShapeDtypeStruct((total, width), jnp.float32),
        mesh=mesh,
        scratch_types=[
            pltpu.VMEM((rows_per_w, _GROWS), jnp.int32),
            pltpu.VMEM((_GROWS, width), jnp.float32),
            pltpu.SemaphoreType.DMA,
        ],
    )
    def k(table_hbm, idx_hbm, out_hbm, idx_v, rows_v, sem):
        c = lax.axis_index("c")
        s = lax.axis_index("s")
        wid = s * 2 + c
        base = wid * rows_per_w
        pltpu.sync_copy(idx_hbm.at[pl.ds(base, rows_per_w)], idx_v)

        def body(j, _):
            pltpu.async_copy(table_hbm.at[idx_v.at[j]], rows_v, sem).wait()
            pltpu.sync_copy(rows_v, out_hbm.at[pl.ds((base + j) * _GROWS, _GROWS)])
            return 0

        lax.fori_loop(0, rows_per_w, body, 0, unroll=False)

    return k(table, idx2d)


_SMLP = 64
_RML = _SMLP * _K


def _mlp_body(g_ref, cxyz_ref, w1a_ref, w1a3_ref, b1a_ref, w1b_ref, b1b_ref,
              wwaf_ref, wwa3_ref, bwa_ref, wwb_ref, bwb_ref, out_ref):
    g = g_ref[...]
    cxyz = cxyz_ref[...]
    ctr1 = jnp.dot(cxyz, w1a3_ref[...], preferred_element_type=jnp.float32)
    ctr2 = jnp.dot(cxyz, wwa3_ref[...], preferred_element_type=jnp.float32)

    def rep(a):
        return jnp.reshape(
            jnp.broadcast_to(a[:, None, :], (_SMLP, _K, a.shape[-1])),
            (_RML, a.shape[-1]))

    t1 = jnp.dot(g, w1a_ref[...], preferred_element_type=jnp.float32)
    h = jnp.maximum(t1 + b1a_ref[...] - rep(ctr1), 0.0)
    fp = jnp.maximum(
        jnp.dot(h, w1b_ref[...], preferred_element_type=jnp.float32) + b1b_ref[...], 0.0)
    fm = jnp.sum(jnp.reshape(fp, (_SMLP, _K, _DOUT)), axis=1) * (1.0 / _K)
    xyzp = g[:, 0:8]
    t2 = (jnp.dot(fp - rep(fm), wwaf_ref[...], preferred_element_type=jnp.float32)
          + jnp.dot(xyzp, wwa3_ref[...], preferred_element_type=jnp.float32)
          - rep(ctr2))
    hw = jnp.maximum(t2 + bwa_ref[...], 0.0)
    logit = jnp.dot(hw, wwb_ref[...], preferred_element_type=jnp.float32) + bwb_ref[...]
    alpha = jax.nn.sigmoid(logit)
    out_ref[...] = jnp.sum(jnp.reshape(alpha * fp, (_SMLP, _K, _DOUT)), axis=1)


def _run_mlp(g, cxyz, w1a, w1a3, b1a, w1b, b1b, wwaf, wwa3, bwa, wwb, bwb):
    n_rows = g.shape[0]
    grid = (n_rows // _RML,)

    def const(shape):
        return pl.BlockSpec(shape, lambda i: tuple(0 for _ in shape))

    return pl.pallas_call(
        _mlp_body,
        grid=grid,
        in_specs=[
            pl.BlockSpec((_RML, 128), lambda i: (i, 0)),
            pl.BlockSpec((_SMLP, 8), lambda i: (i, 0)),
            const(w1a.shape), const(w1a3.shape), const(b1a.shape),
            const(w1b.shape), const(b1b.shape), const(wwaf.shape),
            const(wwa3.shape), const(bwa.shape), const(wwb.shape),
            const(bwb.shape),
        ],
        out_specs=pl.BlockSpec((_SMLP, _DOUT), lambda i: (i, 0)),
        out_shape=jax.ShapeDtypeStruct((n_rows // _K, _DOUT), jnp.float32),
    )(g, cxyz, w1a, w1a3, b1a, w1b, b1b, wwaf, wwa3, bwa, wwb, bwb)


def kernel(xyz, features, W1a, b1a, W1b, b1b, Wwa, bwa, Wwb, bwb):
    B, N, _ = xyz.shape
    C = features.shape[-1]
    x = xyz[:, :, 0]
    y = xyz[:, :, 1]
    z = xyz[:, :, 2]

    nx, ny, nz = _run_fps(x, y, z)
    new_xyz = jnp.stack([nx, ny, nz], axis=-1)

    idx_flat = _run_ball_query_sc(x, y, z, nx, ny, nz)
    idx2d = idx_flat.reshape(-1, _GROWS)

    table = jnp.concatenate(
        [xyz, features, jnp.zeros((B, N, 128 - 3 - C), jnp.float32)],
        axis=-1).reshape(B * N, 128)
    g = _run_gather(table, idx2d)

    cxyz = jnp.concatenate(
        [new_xyz, jnp.zeros((B, _NPOINT, 5), jnp.float32)], axis=-1
    ).reshape(B * _NPOINT, 8)
    w1a = jnp.concatenate([W1a, jnp.zeros((128 - W1a.shape[0], _DOUT), jnp.float32)])
    w1a3 = jnp.concatenate([W1a[:3], jnp.zeros((5, _DOUT), jnp.float32)])
    wwa3 = jnp.concatenate([Wwa[:3], jnp.zeros((5, _DOUT), jnp.float32)])
    wwaf = Wwa[3:]
    f_out = _run_mlp(g, cxyz, w1a, w1a3, b1a[None, :], W1b, b1b[None, :],
                     wwaf, wwa3, bwa[None, :], Wwb, bwb[None, :])
    return (new_xyz, f_out.reshape(B, _NPOINT, _DOUT))


# speedup vs baseline: 1.0271x; 1.0083x over previous
import functools

import jax
import jax.numpy as jnp
from jax import lax
from jax.experimental import pallas as pl
from jax.experimental.pallas import tpu as pltpu
from jax.experimental.pallas import tpu_sc as plsc

_NPOINT = 512
_RADIUS = 0.2
_K = 32
_DOUT = 256
_BIGF = 3.0e38


def _fps_body(x_ref, y_ref, z_ref, nx_ref, ny_ref, nz_ref, dm_ref):
    B, N = x_ref.shape
    x = x_ref[...]
    y = y_ref[...]
    z = z_ref[...]
    lanes = lax.broadcasted_iota(jnp.int32, (B, N), 1)

    fx0 = x[:, 0:1]
    fy0 = y[:, 0:1]
    fz0 = z[:, 0:1]
    dx = x - fx0
    dy = y - fy0
    dz = z - fz0
    dm_ref[...] = dx * dx + dy * dy + dz * dz

    tcols = lax.broadcasted_iota(jnp.int32, (B, 128), 1)

    def body(i, carry):
        tx, ty, tz = carry
        d = dm_ref[...]
        m = jnp.max(d, axis=1, keepdims=True)
        far = jnp.min(jnp.where(d == m, lanes, N), axis=1, keepdims=True)
        sel = lanes == far
        fx = jnp.sum(jnp.where(sel, x, 0.0), axis=1, keepdims=True)
        fy = jnp.sum(jnp.where(sel, y, 0.0), axis=1, keepdims=True)
        fz = jnp.sum(jnp.where(sel, z, 0.0), axis=1, keepdims=True)
        ddx = x - fx
        ddy = y - fy
        ddz = z - fz
        nd = ddx * ddx + ddy * ddy + ddz * ddz
        dm_ref[...] = jnp.minimum(d, nd)
        here = tcols == (i % 128)
        return (jnp.where(here, fx, tx), jnp.where(here, fy, ty),
                jnp.where(here, fz, tz))

    zt = jnp.zeros((B, 128), jnp.float32)
    for p in range(_NPOINT // 128):
        lo = max(p * 128, 1)
        tx, ty, tz = lax.fori_loop(lo, (p + 1) * 128, body, (zt, zt, zt))
        if p == 0:
            first = tcols == 0
            tx = jnp.where(first, fx0, tx)
            ty = jnp.where(first, fy0, ty)
            tz = jnp.where(first, fz0, tz)
        nx_ref[:, p * 128:(p + 1) * 128] = tx
        ny_ref[:, p * 128:(p + 1) * 128] = ty
        nz_ref[:, p * 128:(p + 1) * 128] = tz


def _run_fps(x, y, z):
    B, N = x.shape
    out = jax.ShapeDtypeStruct((B, _NPOINT), jnp.float32)
    return pl.pallas_call(
        _fps_body,
        out_shape=[out, out, out],
        scratch_shapes=[pltpu.VMEM((B, N), jnp.float32)],
    )(x, y, z)


_SBQ = 128


_L = 16
_BIGI = 1 << 30


def _blane(v, lane):
    idx = jnp.full((_L,), lane, jnp.int32)
    return lax.gather(
        v, idx[:, None],
        lax.GatherDimensionNumbers(offset_dims=(), collapsed_slice_dims=(0,),
                                   start_index_map=(0,)),
        (1,), mode=lax.GatherScatterMode.PROMISE_IN_BOUNDS)


_CAP = 1024
_PCAP = 96


def _run_ball_query_sc(xp, yp, zp, nxp, nyp, nzp):
    B, N = xp.shape
    rows_per_w = (B * _NPOINT) // _NW
    per_batch = _NPOINT // rows_per_w
    ngroup = rows_per_w // _L
    nchunk = N // _L
    mesh = plsc.VectorSubcoreMesh(core_axis_name="c", subcore_axis_name="s")
    r2 = _RADIUS ** 2

    @functools.partial(
        pl.kernel,
        out_type=jax.ShapeDtypeStruct((B * _NPOINT * _K,), jnp.int32),
        mesh=mesh,
        scratch_types=[
            pltpu.VMEM((N,), jnp.float32),
            pltpu.VMEM((N,), jnp.float32),
            pltpu.VMEM((N,), jnp.float32),
            pltpu.VMEM((rows_per_w,), jnp.float32),
            pltpu.VMEM((rows_per_w,), jnp.float32),
            pltpu.VMEM((rows_per_w,), jnp.float32),
            pltpu.VMEM((_L * _CAP + _L,), jnp.float32),
            pltpu.VMEM((_L * _CAP + _L,), jnp.int32),
            pltpu.VMEM((_L * _PCAP,), jnp.int32),
            pltpu.VMEM((rows_per_w * _K,), jnp.int32),
            pltpu.SemaphoreType.DMA,
        ],
        compiler_params=pltpu.CompilerParams(needs_layout_passes=False),
    )
    def k(x_hbm, y_hbm, z_hbm, cx_hbm, cy_hbm, cz_hbm, out_hbm,
          xv, yv, zv, cxv, cyv, czv, dbuf, ibuf, pbuf, obuf, sem):
        iota = lax.iota(jnp.int32, _L)
        lane0 = iota == 0
        c = lax.axis_index("c")
        s = lax.axis_index("s")
        wid = s * 2 + c
        b = wid // per_batch
        soff = (wid % per_batch) * rows_per_w
        pltpu.sync_copy(x_hbm.at[b], xv)
        pltpu.sync_copy(y_hbm.at[b], yv)
        pltpu.sync_copy(z_hbm.at[b], zv)
        pltpu.sync_copy(cx_hbm.at[b, pl.ds(soff, rows_per_w)], cxv)
        pltpu.sync_copy(cy_hbm.at[b, pl.ds(soff, rows_per_w)], cyv)
        pltpu.sync_copy(cz_hbm.at[b, pl.ds(soff, rows_per_w)], czv)
        gbase = b * N
        lane_base = iota * _CAP
        plane_base = iota * _PCAP

        def group_body(g, _):
            cx16 = cxv[pl.ds(g * _L, _L)]
            cy16 = cyv[pl.ds(g * _L, _L)]
            cz16 = czv[pl.ds(g * _L, _L)]

            def cand_chunk(t, offv):
                xa = xv[pl.ds(t * _L, _L)]
                ya = yv[pl.ds(t * _L, _L)]
                za = zv[pl.ds(t * _L, _L)]
                for l2 in range(_L):
                    xs = _blane(xa, l2)
                    ys = _blane(ya, l2)
                    zs = _blane(za, l2)
                    dx = cx16 - xs
                    dy = cy16 - ys
                    dz = cz16 - zs
                    d = dx * dx + dy * dy + dz * dz
                    m = d <= r2
                    pos = lane_base + offv
                    jsplat = jnp.full((_L,), t * _L + l2, jnp.int32)
                    plsc.store_scatter(dbuf, [pos], d, mask=m)
                    plsc.store_scatter(ibuf, [pos], jsplat, mask=m)
                    offv = offv + m.astype(jnp.int32)
                return offv

            offv = lax.fori_loop(0, nchunk, cand_chunk,
                                 jnp.zeros((_L,), jnp.int32))

            def pad_chunk(t, offp):
                xa = xv[pl.ds(t * _L, _L)]
                ya = yv[pl.ds(t * _L, _L)]
                za = zv[pl.ds(t * _L, _L)]
                for l2 in range(_L):
                    xs = _blane(xa, l2)
                    ys = _blane(ya, l2)
                    zs = _blane(za, l2)
                    dx = cx16 - xs
                    dy = cy16 - ys
                    dz = cz16 - zs
                    d = dx * dx + dy * dy + dz * dz
                    m = d > r2
                    pos = plane_base + offp
                    jsplat = jnp.full((_L,), t * _L + l2, jnp.int32)
                    plsc.store_scatter(pbuf, [pos], jsplat, mask=m)
                    offp = offp + m.astype(jnp.int32)
                return offp

            lax.fori_loop(0, 4, pad_chunk, jnp.zeros((_L,), jnp.int32))

            def row_fin(r2i, _):
                r = g * _L + r2i
                cnt = jnp.max(_blane(offv, r2i))
                rbase = r2i * _CAP
                pbase = r2i * _PCAP
                obase = r * _K
                plsc.store_scatter(dbuf, [rbase + cnt + iota],
                                   jnp.full((_L,), _BIGF, jnp.float32))

                def few():
                    iv0 = ibuf[pl.ds(rbase, _L)] + gbase
                    iv1 = ibuf[pl.ds(rbase + _L, _L)] + gbase
                    obuf[pl.ds(obase, _L)] = iv0
                    obuf[pl.ds(obase + _L, _L)] = iv1
                    for t in range(2):
                        pv = pbuf[pl.ds(pbase + t * _L, _L)] + gbase
                        ppos = cnt + iota + t * _L
                        plsc.store_scatter(obuf, [ppos + obase], pv,
                                           mask=ppos < _K)

                def many():
                    nch = (cnt + _L) // _L

                    def extract(kk, _):
                        def scan(j, carry):
                            vbest, vpos = carry
                            v = dbuf[pl.ds(rbase + j * _L, _L)]
                            better = v < vbest
                            return (jnp.where(better, v, vbest),
                                    jnp.where(better, j * _L + iota, vpos))

                        vbest, vpos = lax.fori_loop(
                            0, nch, scan,
                            (jnp.full((_L,), _BIGF, jnp.float32),
                             jnp.full((_L,), _BIGI, jnp.int32)))
                        m = jnp.min(vbest)
                        p = jnp.min(jnp.where(vbest == m, vpos, _BIGI))
                        psplat = jnp.full((_L,), p, jnp.int32) + rbase
                        gv = plsc.load_gather(ibuf, [psplat]) + gbase
                        plsc.store_scatter(
                            obuf, [jnp.full((_L,), obase + kk, jnp.int32)],
                            gv, mask=lane0)
                        plsc.store_scatter(
                            dbuf, [psplat],
                            jnp.full((_L,), _BIGF, jnp.float32), mask=lane0)
                        return 0

                    lax.fori_loop(0, _K, extract, 0)

                lax.cond(cnt > _K, many, few)
                return 0

            lax.fori_loop(0, _L, row_fin, 0)
            return 0

        lax.fori_loop(0, ngroup, group_body, 0)
        pltpu.sync_copy(obuf, out_hbm.at[pl.ds(wid * rows_per_w * _K,
                                               rows_per_w * _K)])

    return k(xp, yp, zp, nxp, nyp, nzp)


_NW = 32
_GROWS = 128


def _run_gather(table, idx2d):
    n_idx_rows = idx2d.shape[0]
    rows_per_w = n_idx_rows // _NW
    total = n_idx_rows * _GROWS
    width = table.shape[1]
    mesh = plsc.VectorSubcoreMesh(core_axis_name="c", subcore_axis_name="s")

    @functools.partial(
        pl.kernel,
        out_type=jax.ShapeDtypeStruct((total, width), jnp.float32),
        mesh=mesh,
        scratch_types=[
            pltpu.VMEM((rows_per_w, _GROWS), jnp.int32),
            pltpu.VMEM((_GROWS, width), jnp.float32),
            pltpu.SemaphoreType.DMA,
        ],
    )
    def k(table_hbm, idx_hbm, out_hbm, idx_v, rows_v, sem):
        c = lax.axis_index("c")
        s = lax.axis_index("s")
        wid = s * 2 + c
        base = wid * rows_per_w
        pltpu.sync_copy(idx_hbm.at[pl.ds(base, rows_per_w)], idx_v)

        def body(j, _):
            pltpu.async_copy(table_hbm.at[idx_v.at[j]], rows_v, sem).wait()
            pltpu.sync_copy(rows_v, out_hbm.at[pl.ds((base + j) * _GROWS, _GROWS)])
            return 0

        lax.fori_loop(0, rows_per_w, body, 0, unroll=False)

    return k(table, idx2d)


_SMLP = 128
_RML = _SMLP * _K


def _mlp_body(g_ref, cxyz_ref, w1a_ref, w1a3_ref, b1a_ref, w1b_ref, b1b_ref,
              wwaf_ref, wwa3_ref, bwa_ref, wwb_ref, bwb_ref, out_ref):
    g = g_ref[...]
    cxyz = cxyz_ref[...]
    ctr1 = jnp.dot(cxyz, w1a3_ref[...], preferred_element_type=jnp.float32)
    ctr2 = jnp.dot(cxyz, wwa3_ref[...], preferred_element_type=jnp.float32)

    def rep(a):
        return jnp.reshape(
            jnp.broadcast_to(a[:, None, :], (_SMLP, _K, a.shape[-1])),
            (_RML, a.shape[-1]))

    t1 = jnp.dot(g, w1a_ref[...], preferred_element_type=jnp.float32)
    h = jnp.maximum(t1 + b1a_ref[...] - rep(ctr1), 0.0)
    fp = jnp.maximum(
        jnp.dot(h, w1b_ref[...], preferred_element_type=jnp.float32) + b1b_ref[...], 0.0)
    fm = jnp.sum(jnp.reshape(fp, (_SMLP, _K, _DOUT)), axis=1) * (1.0 / _K)
    xyzp = g[:, 0:8]
    t2 = (jnp.dot(fp - rep(fm), wwaf_ref[...], preferred_element_type=jnp.float32)
          + jnp.dot(xyzp, wwa3_ref[...], preferred_element_type=jnp.float32)
          - rep(ctr2))
    hw = jnp.maximum(t2 + bwa_ref[...], 0.0)
    logit = jnp.dot(hw, wwb_ref[...], preferred_element_type=jnp.float32) + bwb_ref[...]
    alpha = jax.nn.sigmoid(logit)
    out_ref[...] = jnp.sum(jnp.reshape(alpha * fp, (_SMLP, _K, _DOUT)), axis=1)


def _run_mlp(g, cxyz, w1a, w1a3, b1a, w1b, b1b, wwaf, wwa3, bwa, wwb, bwb):
    n_rows = g.shape[0]
    grid = (n_rows // _RML,)

    def const(shape):
        return pl.BlockSpec(shape, lambda i: tuple(0 for _ in shape))

    return pl.pallas_call(
        _mlp_body,
        grid=grid,
        in_specs=[
            pl.BlockSpec((_RML, 128), lambda i: (i, 0)),
            pl.BlockSpec((_SMLP, 8), lambda i: (i, 0)),
            const(w1a.shape), const(w1a3.shape), const(b1a.shape),
            const(w1b.shape), const(b1b.shape), const(wwaf.shape),
            const(wwa3.shape), const(bwa.shape), const(wwb.shape),
            const(bwb.shape),
        ],
        out_specs=pl.BlockSpec((_SMLP, _DOUT), lambda i: (i, 0)),
        out_shape=jax.ShapeDtypeStruct((n_rows // _K, _DOUT), jnp.float32),
    )(g, cxyz, w1a, w1a3, b1a, w1b, b1b, wwaf, wwa3, bwa, wwb, bwb)


def kernel(xyz, features, W1a, b1a, W1b, b1b, Wwa, bwa, Wwb, bwb):
    B, N, _ = xyz.shape
    C = features.shape[-1]
    x = xyz[:, :, 0]
    y = xyz[:, :, 1]
    z = xyz[:, :, 2]

    nx, ny, nz = _run_fps(x, y, z)
    new_xyz = jnp.stack([nx, ny, nz], axis=-1)

    idx_flat = _run_ball_query_sc(x, y, z, nx, ny, nz)
    idx2d = idx_flat.reshape(-1, _GROWS)

    table = jnp.concatenate(
        [xyz, features, jnp.zeros((B, N, 128 - 3 - C), jnp.float32)],
        axis=-1).reshape(B * N, 128)
    g = _run_gather(table, idx2d)

    cxyz = jnp.concatenate(
        [new_xyz, jnp.zeros((B, _NPOINT, 5), jnp.float32)], axis=-1
    ).reshape(B * _NPOINT, 8)
    w1a = jnp.concatenate([W1a, jnp.zeros((128 - W1a.shape[0], _DOUT), jnp.float32)])
    w1a3 = jnp.concatenate([W1a[:3], jnp.zeros((5, _DOUT), jnp.float32)])
    wwa3 = jnp.concatenate([Wwa[:3], jnp.zeros((5, _DOUT), jnp.float32)])
    wwaf = Wwa[3:]
    f_out = _run_mlp(g, cxyz, w1a, w1a3, b1a[None, :], W1b, b1b[None, :],
                     wwaf, wwa3, bwa[None, :], Wwb, bwb[None, :])
    return (new_xyz, f_out.reshape(B, _NPOINT, _DOUT))


# speedup vs baseline: 1.0290x; 1.0019x over previous
import functools

import jax
import jax.numpy as jnp
from jax import lax
from jax.experimental import pallas as pl
from jax.experimental.pallas import tpu as pltpu
from jax.experimental.pallas import tpu_sc as plsc

_NPOINT = 512
_RADIUS = 0.2
_K = 32
_DOUT = 256
_BIGF = 3.0e38


def _fps_body(x_ref, y_ref, z_ref, nx_ref, ny_ref, nz_ref, dm_ref):
    B, N = x_ref.shape
    x = x_ref[...]
    y = y_ref[...]
    z = z_ref[...]
    lanes = lax.broadcasted_iota(jnp.int32, (B, N), 1)

    fx0 = x[:, 0:1]
    fy0 = y[:, 0:1]
    fz0 = z[:, 0:1]
    dx = x - fx0
    dy = y - fy0
    dz = z - fz0
    dm_ref[...] = dx * dx + dy * dy + dz * dz

    tcols = lax.broadcasted_iota(jnp.int32, (B, 128), 1)

    def body(i, carry):
        tx, ty, tz = carry
        d = dm_ref[...]
        m = jnp.max(d, axis=1, keepdims=True)
        far = jnp.min(jnp.where(d == m, lanes, N), axis=1, keepdims=True)
        sel = lanes == far
        fx = jnp.sum(jnp.where(sel, x, 0.0), axis=1, keepdims=True)
        fy = jnp.sum(jnp.where(sel, y, 0.0), axis=1, keepdims=True)
        fz = jnp.sum(jnp.where(sel, z, 0.0), axis=1, keepdims=True)
        ddx = x - fx
        ddy = y - fy
        ddz = z - fz
        nd = ddx * ddx + ddy * ddy + ddz * ddz
        dm_ref[...] = jnp.minimum(d, nd)
        here = tcols == (i % 128)
        return (jnp.where(here, fx, tx), jnp.where(here, fy, ty),
                jnp.where(here, fz, tz))

    zt = jnp.zeros((B, 128), jnp.float32)
    for p in range(_NPOINT // 128):
        lo = max(p * 128, 1)
        tx, ty, tz = lax.fori_loop(lo, (p + 1) * 128, body, (zt, zt, zt))
        if p == 0:
            first = tcols == 0
            tx = jnp.where(first, fx0, tx)
            ty = jnp.where(first, fy0, ty)
            tz = jnp.where(first, fz0, tz)
        nx_ref[:, p * 128:(p + 1) * 128] = tx
        ny_ref[:, p * 128:(p + 1) * 128] = ty
        nz_ref[:, p * 128:(p + 1) * 128] = tz


def _run_fps(x, y, z):
    B, N = x.shape
    out = jax.ShapeDtypeStruct((B, _NPOINT), jnp.float32)
    return pl.pallas_call(
        _fps_body,
        out_shape=[out, out, out],
        scratch_shapes=[pltpu.VMEM((B, N), jnp.float32)],
    )(x, y, z)


_L = 16
_BIGI = 1 << 30


def _blane(v, lane):
    idx = jnp.full((_L,), lane, jnp.int32)
    return lax.gather(
        v, idx[:, None],
        lax.GatherDimensionNumbers(offset_dims=(), collapsed_slice_dims=(0,),
                                   start_index_map=(0,)),
        (1,), mode=lax.GatherScatterMode.PROMISE_IN_BOUNDS)


_CAP = 1024
_PCAP = 96


def _run_ball_query_sc(xp, yp, zp, nxp, nyp, nzp):
    B, N = xp.shape
    rows_per_w = (B * _NPOINT) // _NW
    per_batch = _NPOINT // rows_per_w
    ngroup = rows_per_w // _L
    nchunk = N // _L
    mesh = plsc.VectorSubcoreMesh(core_axis_name="c", subcore_axis_name="s")
    r2 = _RADIUS ** 2

    @functools.partial(
        pl.kernel,
        out_type=jax.ShapeDtypeStruct((B * _NPOINT * _K,), jnp.int32),
        mesh=mesh,
        scratch_types=[
            pltpu.VMEM((N,), jnp.float32),
            pltpu.VMEM((N,), jnp.float32),
            pltpu.VMEM((N,), jnp.float32),
            pltpu.VMEM((rows_per_w,), jnp.float32),
            pltpu.VMEM((rows_per_w,), jnp.float32),
            pltpu.VMEM((rows_per_w,), jnp.float32),
            pltpu.VMEM((_L * _CAP + _L,), jnp.float32),
            pltpu.VMEM((_L * _CAP + _L,), jnp.int32),
            pltpu.VMEM((_L * _PCAP,), jnp.int32),
            pltpu.VMEM((rows_per_w * _K,), jnp.int32),
            pltpu.SemaphoreType.DMA,
        ],
        compiler_params=pltpu.CompilerParams(needs_layout_passes=False),
    )
    def k(x_hbm, y_hbm, z_hbm, cx_hbm, cy_hbm, cz_hbm, out_hbm,
          xv, yv, zv, cxv, cyv, czv, dbuf, ibuf, pbuf, obuf, sem):
        iota = lax.iota(jnp.int32, _L)
        lane0 = iota == 0
        c = lax.axis_index("c")
        s = lax.axis_index("s")
        wid = s * 2 + c
        b = wid // per_batch
        soff = (wid % per_batch) * rows_per_w
        pltpu.sync_copy(x_hbm.at[b], xv)
        pltpu.sync_copy(y_hbm.at[b], yv)
        pltpu.sync_copy(z_hbm.at[b], zv)
        pltpu.sync_copy(cx_hbm.at[b, pl.ds(soff, rows_per_w)], cxv)
        pltpu.sync_copy(cy_hbm.at[b, pl.ds(soff, rows_per_w)], cyv)
        pltpu.sync_copy(cz_hbm.at[b, pl.ds(soff, rows_per_w)], czv)
        gbase = b * N
        lane_base = iota * _CAP
        plane_base = iota * _PCAP

        def group_body(g, _):
            cx16 = cxv[pl.ds(g * _L, _L)]
            cy16 = cyv[pl.ds(g * _L, _L)]
            cz16 = czv[pl.ds(g * _L, _L)]

            def cand_chunk(t, offv):
                xa = xv[pl.ds(t * _L, _L)]
                ya = yv[pl.ds(t * _L, _L)]
                za = zv[pl.ds(t * _L, _L)]
                for l2 in range(_L):
                    xs = _blane(xa, l2)
                    ys = _blane(ya, l2)
                    zs = _blane(za, l2)
                    dx = cx16 - xs
                    dy = cy16 - ys
                    dz = cz16 - zs
                    d = dx * dx + dy * dy + dz * dz
                    m = d <= r2
                    pos = lane_base + offv
                    jsplat = jnp.full((_L,), t * _L + l2, jnp.int32)
                    plsc.store_scatter(dbuf, [pos], d, mask=m)
                    plsc.store_scatter(ibuf, [pos], jsplat, mask=m)
                    offv = offv + m.astype(jnp.int32)
                return offv

            offv = lax.fori_loop(0, nchunk, cand_chunk,
                                 jnp.zeros((_L,), jnp.int32))

            def pad_chunk(t, offp):
                xa = xv[pl.ds(t * _L, _L)]
                ya = yv[pl.ds(t * _L, _L)]
                za = zv[pl.ds(t * _L, _L)]
                for l2 in range(_L):
                    xs = _blane(xa, l2)
                    ys = _blane(ya, l2)
                    zs = _blane(za, l2)
                    dx = cx16 - xs
                    dy = cy16 - ys
                    dz = cz16 - zs
                    d = dx * dx + dy * dy + dz * dz
                    m = d > r2
                    pos = plane_base + offp
                    jsplat = jnp.full((_L,), t * _L + l2, jnp.int32)
                    plsc.store_scatter(pbuf, [pos], jsplat, mask=m)
                    offp = offp + m.astype(jnp.int32)
                return offp

            lax.fori_loop(0, 4, pad_chunk, jnp.zeros((_L,), jnp.int32))

            def row_fin(r2i, _):
                r = g * _L + r2i
                cnt = jnp.max(_blane(offv, r2i))
                rbase = r2i * _CAP
                pbase = r2i * _PCAP
                obase = r * _K
                plsc.store_scatter(dbuf, [rbase + cnt + iota],
                                   jnp.full((_L,), _BIGF, jnp.float32))

                def few():
                    iv0 = ibuf[pl.ds(rbase, _L)] + gbase
                    iv1 = ibuf[pl.ds(rbase + _L, _L)] + gbase
                    obuf[pl.ds(obase, _L)] = iv0
                    obuf[pl.ds(obase + _L, _L)] = iv1
                    for t in range(2):
                        pv = pbuf[pl.ds(pbase + t * _L, _L)] + gbase
                        ppos = cnt + iota + t * _L
                        plsc.store_scatter(obuf, [ppos + obase], pv,
                                           mask=ppos < _K)

                def many():
                    nch = (cnt + _L) // _L

                    def extract(kk, _):
                        def scan(j, carry):
                            vbest, vpos = carry
                            v = dbuf[pl.ds(rbase + j * _L, _L)]
                            better = v < vbest
                            return (jnp.where(better, v, vbest),
                                    jnp.where(better, j * _L + iota, vpos))

                        vbest, vpos = lax.fori_loop(
                            0, nch, scan,
                            (jnp.full((_L,), _BIGF, jnp.float32),
                             jnp.full((_L,), _BIGI, jnp.int32)))
                        m = jnp.min(vbest)
                        p = jnp.min(jnp.where(vbest == m, vpos, _BIGI))
                        psplat = jnp.full((_L,), p, jnp.int32) + rbase
                        gv = plsc.load_gather(ibuf, [psplat]) + gbase
                        plsc.store_scatter(
                            obuf, [jnp.full((_L,), obase + kk, jnp.int32)],
                            gv, mask=lane0)
                        plsc.store_scatter(
                            dbuf, [psplat],
                            jnp.full((_L,), _BIGF, jnp.float32), mask=lane0)
                        return 0

                    lax.fori_loop(0, _K, extract, 0)

                lax.cond(cnt > _K, many, few)
                return 0

            lax.fori_loop(0, _L, row_fin, 0)
            return 0

        lax.fori_loop(0, ngroup, group_body, 0)
        pltpu.sync_copy(obuf, out_hbm.at[pl.ds(wid * rows_per_w * _K,
                                               rows_per_w * _K)])

    return k(xp, yp, zp, nxp, nyp, nzp)


_NW = 32
_GROWS = 128


def _run_gather(table, idx2d):
    n_idx_rows = idx2d.shape[0]
    rows_per_w = n_idx_rows // _NW
    total = n_idx_rows * _GROWS
    width = table.shape[1]
    mesh = plsc.VectorSubcoreMesh(core_axis_name="c", subcore_axis_name="s")

    @functools.partial(
        pl.kernel,
        out_type=jax.ShapeDtypeStruct((total, width), jnp.float32),
        mesh=mesh,
        scratch_types=[
            pltpu.VMEM((rows_per_w, _GROWS), jnp.int32),
            pltpu.VMEM((_GROWS, width), jnp.float32),
            pltpu.SemaphoreType.DMA,
        ],
    )
    def k(table_hbm, idx_hbm, out_hbm, idx_v, rows_v, sem):
        c = lax.axis_index("c")
        s = lax.axis_index("s")
        wid = s * 2 + c
        base = wid * rows_per_w
        pltpu.sync_copy(idx_hbm.at[pl.ds(base, rows_per_w)], idx_v)

        def body(j, _):
            pltpu.async_copy(table_hbm.at[idx_v.at[j]], rows_v, sem).wait()
            pltpu.sync_copy(rows_v, out_hbm.at[pl.ds((base + j) * _GROWS, _GROWS)])
            return 0

        lax.fori_loop(0, rows_per_w, body, 0, unroll=False)

    return k(table, idx2d)


_SMLP = 128
_RML = _SMLP * _K


def _mlp_body(g_ref, cxyz_ref, w1a_ref, w1a3_ref, b1a_ref, w1b_ref, b1b_ref,
              wwaf_ref, wwa3_ref, bwa_ref, wwb_ref, bwb_ref, out_ref):
    g = g_ref[...]
    cxyz = cxyz_ref[...]
    ctr1 = jnp.dot(cxyz, w1a3_ref[...], preferred_element_type=jnp.float32)
    ctr2 = jnp.dot(cxyz, wwa3_ref[...], preferred_element_type=jnp.float32)

    def rep(a):
        return jnp.reshape(
            jnp.broadcast_to(a[:, None, :], (_SMLP, _K, a.shape[-1])),
            (_RML, a.shape[-1]))

    t1 = jnp.dot(g, w1a_ref[...], preferred_element_type=jnp.float32)
    h = jnp.maximum(t1 + b1a_ref[...] - rep(ctr1), 0.0)
    fp = jnp.maximum(
        jnp.dot(h, w1b_ref[...], preferred_element_type=jnp.float32) + b1b_ref[...], 0.0)
    fm = jnp.sum(jnp.reshape(fp, (_SMLP, _K, _DOUT)), axis=1) * (1.0 / _K)
    xyzp = g[:, 0:8]
    t2 = (jnp.dot(fp - rep(fm), wwaf_ref[...], preferred_element_type=jnp.float32)
          + jnp.dot(xyzp, wwa3_ref[...], preferred_element_type=jnp.float32)
          - rep(ctr2))
    hw = jnp.maximum(t2 + bwa_ref[...], 0.0)
    logit = jnp.dot(hw, wwb_ref[...], preferred_element_type=jnp.float32) + bwb_ref[...]
    alpha = jax.nn.sigmoid(logit)
    out_ref[...] = jnp.sum(jnp.reshape(alpha * fp, (_SMLP, _K, _DOUT)), axis=1)


def _run_mlp(g, cxyz, w1a, w1a3, b1a, w1b, b1b, wwaf, wwa3, bwa, wwb, bwb):
    n_rows = g.shape[0]
    grid = (n_rows // _RML,)

    def const(shape):
        return pl.BlockSpec(shape, lambda i: tuple(0 for _ in shape))

    return pl.pallas_call(
        _mlp_body,
        grid=grid,
        in_specs=[
            pl.BlockSpec((_RML, 128), lambda i: (i, 0)),
            pl.BlockSpec((_SMLP, 8), lambda i: (i, 0)),
            const(w1a.shape), const(w1a3.shape), const(b1a.shape),
            const(w1b.shape), const(b1b.shape), const(wwaf.shape),
            const(wwa3.shape), const(bwa.shape), const(wwb.shape),
            const(bwb.shape),
        ],
        out_specs=pl.BlockSpec((_SMLP, _DOUT), lambda i: (i, 0)),
        out_shape=jax.ShapeDtypeStruct((n_rows // _K, _DOUT), jnp.float32),
    )(g, cxyz, w1a, w1a3, b1a, w1b, b1b, wwaf, wwa3, bwa, wwb, bwb)


def kernel(xyz, features, W1a, b1a, W1b, b1b, Wwa, bwa, Wwb, bwb):
    B, N, _ = xyz.shape
    C = features.shape[-1]
    x = xyz[:, :, 0]
    y = xyz[:, :, 1]
    z = xyz[:, :, 2]

    nx, ny, nz = _run_fps(x, y, z)
    new_xyz = jnp.stack([nx, ny, nz], axis=-1)

    idx_flat = _run_ball_query_sc(x, y, z, nx, ny, nz)
    idx2d = idx_flat.reshape(-1, _GROWS)

    table = jnp.concatenate(
        [xyz, features, jnp.zeros((B, N, 128 - 3 - C), jnp.float32)],
        axis=-1).reshape(B * N, 128)
    g = _run_gather(table, idx2d)

    cxyz = jnp.concatenate(
        [new_xyz, jnp.zeros((B, _NPOINT, 5), jnp.float32)], axis=-1
    ).reshape(B * _NPOINT, 8)
    w1a = jnp.concatenate([W1a, jnp.zeros((128 - W1a.shape[0], _DOUT), jnp.float32)])
    w1a3 = jnp.concatenate([W1a[:3], jnp.zeros((5, _DOUT), jnp.float32)])
    wwa3 = jnp.concatenate([Wwa[:3], jnp.zeros((5, _DOUT), jnp.float32)])
    wwaf = Wwa[3:]
    f_out = _run_mlp(g, cxyz, w1a, w1a3, b1a[None, :], W1b, b1b[None, :],
                     wwaf, wwa3, bwa[None, :], Wwb, bwb[None, :])
    return (new_xyz, f_out.reshape(B, _NPOINT, _DOUT))
